# baseline (speedup 1.0000x reference)
import functools

import jax
import jax.numpy as jnp
from jax.experimental import pallas as pl
from jax.experimental.pallas import tpu as pltpu

_L = 3
_C0 = 3
_HID = 64
_HW = 128
_BG = 2


def _leaky(v):
    return jnp.maximum(v, 0.1 * v)


def _pool_lanes(z, hid):
    m01 = jnp.maximum(z[:, 0:hid], z[:, hid:2 * hid])
    m23 = jnp.maximum(z[:, 2 * hid:3 * hid], z[:, 3 * hid:4 * hid])
    return jnp.maximum(m01, m23)


def _fused_net_kernel(p4_ref, w4_ref, s4_ref, b4_ref,
                      wt4_ref, st4_ref, bt4_ref, out_ref,
                      ypad_ref, lhs_ref, *, bg, hid):
    s4 = s4_ref[0]
    b4 = b4_ref[0]
    for g in range(bg):
        for r in range(4):
            rows = pl.ds(g * 4096 + r * 1024, 1024)
            acc = jnp.dot(p4_ref[0, rows, :], w4_ref[0],
                          preferred_element_type=jnp.float32)
            z = acc * s4 + b4
            y = _leaky(_pool_lanes(z, hid))
            ypad_ref[g, 1 + r * 16:17 + r * 16, 1:65, :] = (
                y.reshape(16, 64, hid))

    size = 64
    for k in range(6):
        sh = size // 2
        p = sh * sh
        for g in range(bg):
            zrow = jnp.zeros((size + 2, hid), jnp.float32)
            zcol = jnp.zeros((size, 1, hid), jnp.float32)
            ypad_ref[g, 0, 0:size + 2, :] = zrow
            ypad_ref[g, size + 1, 0:size + 2, :] = zrow
            ypad_ref[g, 1:size + 1, 0, :] = zcol[:, 0]
            ypad_ref[g, 1:size + 1, size + 1, :] = zcol[:, 0]
        for g in range(bg):
            for t in range(16):
                iy, ix = divmod(t, 4)
                sl = ypad_ref[g, pl.ds(iy, sh, 2), pl.ds(ix, sh, 2), :]
                lhs_ref[g * p:(g + 1) * p, t * hid:(t + 1) * hid] = (
                    sl.reshape(p, hid).astype(jnp.bfloat16))
        acc = jnp.dot(lhs_ref[0:bg * p, :], wt4_ref[0, k],
                      preferred_element_type=jnp.float32)
        z = acc * st4_ref[0, k] + bt4_ref[0, k]
        y = _leaky(_pool_lanes(z, hid))
        if k < 5:
            for g in range(bg):
                ypad_ref[g, 1:sh + 1, 1:sh + 1, :] = (
                    y[g * p:(g + 1) * p].reshape(sh, sh, hid))
        else:
            out_ref[0] = y
        size = sh


def _expand_w4(w9, hid):
    parts = [jnp.pad(w9, ((py, 1 - py), (px, 1 - px), (0, 0), (0, 0)))
             for py in (0, 1) for px in (0, 1)]
    w4 = jnp.stack(parts, axis=-2)
    cin = w9.shape[2]
    return w4.reshape(16 * cin, 4 * hid)


def kernel(x, w0_0, s0_0, b0_0, wt_0, st_0, bt_0,
           w0_1, s0_1, b0_1, wt_1, st_1, bt_1,
           w0_2, s0_2, b0_2, wt_2, st_2, bt_2, wc, bc):
    L, C0, HID, HW, BG = _L, _C0, _HID, _HW, _BG
    B = x.shape[0]
    half = HW // 2

    x5 = x.reshape(B, L, C0, HW, HW)
    xp = jnp.pad(x5, ((0, 0), (0, 0), (0, 0), (1, 1), (1, 1)))
    taps = [xp[:, :, :, iy:iy + HW:2, ix:ix + HW:2]
            for iy in range(4) for ix in range(4)]
    p4 = jnp.stack(taps, axis=-1)
    p4 = p4.transpose(1, 0, 3, 4, 5, 2)
    p4 = p4.reshape(L * B // BG, BG * half * half, 16 * C0).astype(jnp.bfloat16)

    w4 = jnp.stack([_expand_w4(w.reshape(3, 3, C0, HID), HID)
                    for w in (w0_0, w0_1, w0_2)]).astype(jnp.bfloat16)
    s4 = jnp.stack([jnp.tile(s, (1, 4)) for s in (s0_0, s0_1, s0_2)])
    b4 = jnp.stack([jnp.tile(b, (1, 4)) for b in (b0_0, b0_1, b0_2)])
    wt4 = jnp.stack(
        [jnp.stack([_expand_w4(wt[k].reshape(3, 3, HID, HID), HID)
                    for k in range(6)]) for wt in (wt_0, wt_1, wt_2)]
    ).astype(jnp.bfloat16)
    st4 = jnp.stack([jnp.tile(s, (1, 1, 4)) for s in (st_0, st_1, st_2)])
    bt4 = jnp.stack([jnp.tile(b, (1, 1, 4)) for b in (bt_0, bt_1, bt_2)])

    steps_per_level = B // BG
    feats = pl.pallas_call(
        functools.partial(_fused_net_kernel, bg=BG, hid=HID),
        out_shape=jax.ShapeDtypeStruct((L * steps_per_level, BG, HID),
                                       jnp.float32),
        grid=(L * steps_per_level,),
        in_specs=[
            pl.BlockSpec((1, BG * half * half, 16 * C0), lambda i: (i, 0, 0)),
            pl.BlockSpec((1, 16 * C0, 4 * HID),
                         lambda i, n=steps_per_level: (i // n, 0, 0)),
            pl.BlockSpec((1, 1, 4 * HID),
                         lambda i, n=steps_per_level: (i // n, 0, 0)),
            pl.BlockSpec((1, 1, 4 * HID),
                         lambda i, n=steps_per_level: (i // n, 0, 0)),
            pl.BlockSpec((1, 6, 16 * HID, 4 * HID),
                         lambda i, n=steps_per_level: (i // n, 0, 0, 0)),
            pl.BlockSpec((1, 6, 1, 4 * HID),
                         lambda i, n=steps_per_level: (i // n, 0, 0, 0)),
            pl.BlockSpec((1, 6, 1, 4 * HID),
                         lambda i, n=steps_per_level: (i // n, 0, 0, 0)),
        ],
        out_specs=pl.BlockSpec((1, BG, HID), lambda i: (i, 0, 0)),
        scratch_shapes=[
            pltpu.VMEM((BG, 66, 66, HID), jnp.float32),
            pltpu.VMEM((BG * 1024, 16 * HID), jnp.bfloat16),
        ],
        compiler_params=pltpu.CompilerParams(
            dimension_semantics=("parallel",),
            vmem_limit_bytes=64 * 1024 * 1024),
    )(p4, w4, s4, b4, wt4, st4, bt4)

    f = feats.reshape(L, B, HID).transpose(1, 0, 2).reshape(B, L * HID)
    out = jnp.dot(f, wc, precision=jax.lax.Precision.HIGHEST) + bc
    return out.reshape(B, 1, 1, 1)
